# baseline (speedup 1.0000x reference)
import functools

import jax
import jax.numpy as jnp
from jax import lax
from jax.experimental import pallas as pl
from jax.experimental.pallas import tpu as pltpu
from jax.experimental.pallas import tpu_sc as plsc

N = 1_000_000
D = 64
NC = 2
NS = 16
NW = NC * NS
L = 16

CHUNK = 400
PAD = 17
CBLK = 4
MAIN_PER_TILE = 78
PAIRS = MAIN_PER_TILE // 2
ROWS_PER_TILE = CHUNK * MAIN_PER_TILE
MAIN_ROWS = ROWS_PER_TILE * NW
TAIL_CHUNKS = (N - MAIN_ROWS) // CHUNK
GROUPS = CHUNK // L


def _rsqrt16(x):
    xi = plsc.bitcast(x, jnp.int32)
    y = plsc.bitcast(jnp.int32(0x5F3759DF) - (xi >> 1), jnp.float32)
    xh = x * jnp.float32(-0.5)
    for _ in range(3):
        y = y * (jnp.float32(1.5) + xh * y * y)
    return y


def _b16r(x):
    xi = plsc.bitcast(x, jnp.int32)
    return plsc.bitcast((xi + jnp.int32(0x8000)) & jnp.int32(-65536),
                        jnp.float32)


def _sc_body(cand_hbm, slots_hbm, scores_hbm, pmax_hbm, pidx_hbm,
             cand_v, ia0, ia1, ia2, ia3, ib0, ib1, ib2, ib3,
             sc_a, sc_b, rows_v, mvec, ivec,
             sem_ia, sem_ib, sem_oa, sem_ob):
    in_a = (ia0, ia1, ia2, ia3)
    in_b = (ib0, ib1, ib2, ib3)
    c = lax.axis_index("c")
    s = lax.axis_index("s")
    wid = s * NC + c

    pltpu.sync_copy(cand_hbm, cand_v)
    c0 = cand_v[pl.ds(0, L)]
    c1 = cand_v[pl.ds(L, L)]
    c2 = cand_v[pl.ds(2 * L, L)]
    c3 = cand_v[pl.ds(3 * L, L)]
    cn2 = jnp.sum(c0 * c0 + c1 * c1 + c2 * c2 + c3 * c3)
    inv_c = _rsqrt16(jnp.full((L,), jnp.maximum(cn2, jnp.float32(1e-30)),
                              jnp.float32))
    cn = (c0 * inv_c, c1 * inv_c, c2 * inv_c, c3 * inv_c)

    mvec[...] = jnp.full((L,), -jnp.inf, jnp.float32)
    ivec[...] = jnp.zeros((L,), jnp.int32)
    iota = lax.iota(jnp.int32, L)
    cols = [jnp.full((L,), c, jnp.int32) for c in range(L)]

    def compute_chunk(bufs, sc_v, row0):
        def group(g, carry):
            rvec = iota + g * L
            acc_d = jnp.zeros((L,), jnp.float32)
            acc_n = jnp.zeros((L,), jnp.float32)
            for j in range(D):
                w = plsc.load_gather(bufs[j // L], [rvec, cols[j % L]])
                acc_d = acc_d + w * cn[j // L][j % L]
                acc_n = acc_n + w * w
            inv = _rsqrt16(jnp.maximum(acc_n, jnp.float32(1e-30)))
            sc16 = acc_d * inv
            sc_v[pl.ds(g * L, L)] = sc16
            idx16 = iota + (row0 + g * L)
            m = mvec[...]
            better = sc16 > m
            mvec[...] = jnp.where(better, sc16, m)
            ivec[...] = jnp.where(better, idx16, ivec[...])
            return carry

        lax.fori_loop(0, GROUPS, group, 0)

    def in_copies(row0, bufs, sem):
        return [pltpu.make_async_copy(
            slots_hbm.at[pl.ds(row0, CHUNK), pl.ds(L * k, L)],
            bufs[k].at[:, pl.ds(0, L)], sem) for k in range(CBLK)]

    def in_start(row0, bufs, sem):
        for cp in in_copies(row0, bufs, sem):
            cp.start()

    def in_wait(row0, bufs, sem):
        for cp in in_copies(row0, bufs, sem):
            cp.wait()

    def out_copy(row0, buf, sem):
        return pltpu.make_async_copy(
            buf, scores_hbm.at[pl.ds(row0, CHUNK)], sem)

    base = wid * ROWS_PER_TILE
    in_start(base, in_a, sem_ia)

    def pair(p, carry):
        r0 = base + (2 * p) * CHUNK
        in_wait(r0, in_a, sem_ia)
        in_start(r0 + CHUNK, in_b, sem_ib)

        @pl.when(p > 0)
        def _():
            out_copy(r0 - 2 * CHUNK, sc_a, sem_oa).wait()

        compute_chunk(in_a, sc_a, r0)
        out_copy(r0, sc_a, sem_oa).start()

        in_wait(r0 + CHUNK, in_b, sem_ib)

        @pl.when(p < PAIRS - 1)
        def _():
            in_start(r0 + 2 * CHUNK, in_a, sem_ia)

        @pl.when(p > 0)
        def _():
            out_copy(r0 - CHUNK, sc_b, sem_ob).wait()

        compute_chunk(in_b, sc_b, r0 + CHUNK)
        out_copy(r0 + CHUNK, sc_b, sem_ob).start()
        return carry

    lax.fori_loop(0, PAIRS, pair, 0)
    out_copy(base + (MAIN_PER_TILE - 2) * CHUNK, sc_a, sem_oa).wait()
    out_copy(base + (MAIN_PER_TILE - 1) * CHUNK, sc_b, sem_ob).wait()

    @pl.when(wid < TAIL_CHUNKS)
    def _():
        row0 = MAIN_ROWS + wid * CHUNK
        in_start(row0, in_a, sem_ia)
        in_wait(row0, in_a, sem_ia)
        compute_chunk(in_a, sc_a, row0)
        pltpu.sync_copy(sc_a, scores_hbm.at[pl.ds(row0, CHUNK)])

    cb = tuple(_b16r(x) for x in cn)
    pltpu.async_copy(slots_hbm.at[ivec], rows_v, sem_ia).wait()
    ivv = ivec[...]
    best_s = jnp.float32(-jnp.inf)
    best_i = jnp.int32(0)
    for r in range(L):
        v0 = rows_v[r, pl.ds(0, L)]
        v1 = rows_v[r, pl.ds(L, L)]
        v2 = rows_v[r, pl.ds(2 * L, L)]
        v3 = rows_v[r, pl.ds(3 * L, L)]
        nv = v0 * v0 + v1 * v1 + v2 * v2 + v3 * v3
        n2 = jnp.cumsum(nv)[L - 1]
        invv = _rsqrt16(jnp.full((L,), jnp.maximum(n2, jnp.float32(1e-30)),
                                 jnp.float32))
        u0 = _b16r(v0 * invv)
        u1 = _b16r(v1 * invv)
        u2 = _b16r(v2 * invv)
        u3 = _b16r(v3 * invv)
        sv = u0 * cb[0] + u1 * cb[1] + u2 * cb[2] + u3 * cb[3]
        s_r = jnp.cumsum(sv)[L - 1]
        i_r = ivv[r]
        take = (s_r > best_s) | ((s_r == best_s) & (i_r < best_i))
        best_s = jnp.where(take, s_r, best_s)
        best_i = jnp.where(take, i_r, best_i)
    mvec[...] = jnp.full((L,), best_s, jnp.float32)
    ivec[...] = jnp.full((L,), best_i, jnp.int32)

    pltpu.sync_copy(mvec, pmax_hbm.at[wid])
    pltpu.sync_copy(ivec, pidx_hbm.at[wid])


def _merge_body(pm_ref, pi_ref, o_ref):
    m = pm_ref[...]
    i = pi_ref[...]
    best = jnp.max(m)
    o_ref[0, 0] = jnp.min(jnp.where(m == best, i, jnp.int32(2147483647)))


def _merge(pmax, pidx):
    return pl.pallas_call(
        _merge_body,
        out_shape=jax.ShapeDtypeStruct((1, 1), jnp.int32),
        out_specs=pl.BlockSpec(memory_space=pltpu.SMEM),
    )(pmax, pidx)


@jax.jit
def kernel(candidate, slot_embeds):
    mesh = plsc.VectorSubcoreMesh(core_axis_name="c", subcore_axis_name="s")
    sc_call = pl.kernel(
        _sc_body,
        out_type=[
            jax.ShapeDtypeStruct((N,), jnp.float32),
            jax.ShapeDtypeStruct((NW, L), jnp.float32),
            jax.ShapeDtypeStruct((NW, L), jnp.int32),
        ],
        scratch_types=[
            pltpu.VMEM((D,), jnp.float32),
            pltpu.VMEM((CHUNK, PAD), jnp.float32),
            pltpu.VMEM((CHUNK, PAD), jnp.float32),
            pltpu.VMEM((CHUNK, PAD), jnp.float32),
            pltpu.VMEM((CHUNK, PAD), jnp.float32),
            pltpu.VMEM((CHUNK, PAD), jnp.float32),
            pltpu.VMEM((CHUNK, PAD), jnp.float32),
            pltpu.VMEM((CHUNK, PAD), jnp.float32),
            pltpu.VMEM((CHUNK, PAD), jnp.float32),
            pltpu.VMEM((CHUNK,), jnp.float32),
            pltpu.VMEM((CHUNK,), jnp.float32),
            pltpu.VMEM((L, D), jnp.float32),
            pltpu.VMEM((L,), jnp.float32),
            pltpu.VMEM((L,), jnp.int32),
            pltpu.SemaphoreType.DMA,
            pltpu.SemaphoreType.DMA,
            pltpu.SemaphoreType.DMA,
            pltpu.SemaphoreType.DMA,
        ],
        mesh=mesh,
        compiler_params=pltpu.CompilerParams(needs_layout_passes=False,
                                             use_tc_tiling_on_sc=False),
    )
    scores, pmax, pidx = sc_call(candidate, slot_embeds)
    best = _merge(pmax, pidx)[0, 0]
    return scores, best


# speedup vs baseline: 1.1045x; 1.1045x over previous
import functools

import jax
import jax.numpy as jnp
from jax import lax
from jax.experimental import pallas as pl
from jax.experimental.pallas import tpu as pltpu
from jax.experimental.pallas import tpu_sc as plsc

N = 1_000_000
D = 64
NC = 2
NS = 16
NW = NC * NS
L = 16

CHUNK = 400
PAD = 17
CBLK = 4
MAIN_PER_TILE = 78
PAIRS = MAIN_PER_TILE // 2
ROWS_PER_TILE = CHUNK * MAIN_PER_TILE
MAIN_ROWS = ROWS_PER_TILE * NW
TAIL_CHUNKS = (N - MAIN_ROWS) // CHUNK
GROUPS = CHUNK // L


def _rsqrt16(x):
    xi = plsc.bitcast(x, jnp.int32)
    y = plsc.bitcast(jnp.int32(0x5F3759DF) - (xi >> 1), jnp.float32)
    xh = x * jnp.float32(-0.5)
    for _ in range(3):
        y = y * (jnp.float32(1.5) + xh * y * y)
    return y


def _b16r(x):
    xi = plsc.bitcast(x, jnp.int32)
    return plsc.bitcast((xi + jnp.int32(0x8000)) & jnp.int32(-65536),
                        jnp.float32)


def _sc_body(cand_hbm, slots_hbm, scores_hbm, pmax_hbm, pidx_hbm,
             cand_v, ia0, ia1, ia2, ia3, ib0, ib1, ib2, ib3,
             sc_a, sc_b, rows_v, mvec, ivec,
             sem_ia, sem_ib, sem_oa, sem_ob):
    in_a = (ia0, ia1, ia2, ia3)
    in_b = (ib0, ib1, ib2, ib3)
    c = lax.axis_index("c")
    s = lax.axis_index("s")
    wid = s * NC + c

    pltpu.sync_copy(cand_hbm, cand_v)
    c0 = cand_v[pl.ds(0, L)]
    c1 = cand_v[pl.ds(L, L)]
    c2 = cand_v[pl.ds(2 * L, L)]
    c3 = cand_v[pl.ds(3 * L, L)]
    cn2 = jnp.sum(c0 * c0 + c1 * c1 + c2 * c2 + c3 * c3)
    inv_c = _rsqrt16(jnp.full((L,), jnp.maximum(cn2, jnp.float32(1e-30)),
                              jnp.float32))
    cn = (c0 * inv_c, c1 * inv_c, c2 * inv_c, c3 * inv_c)

    mvec[...] = jnp.full((L,), -jnp.inf, jnp.float32)
    ivec[...] = jnp.zeros((L,), jnp.int32)
    iota = lax.iota(jnp.int32, L)
    zeros_i = iota * 0

    def compute_chunk(bufs, sc_v, row0, unroll):
        def group(g, mv_iv):
            mv, iv = mv_iv
            rvec = iota + g * L
            accd = [jnp.zeros((L,), jnp.float32) for _ in range(CBLK)]
            accn = [jnp.zeros((L,), jnp.float32) for _ in range(CBLK)]
            colv = zeros_i
            for c in range(L):
                for k in range(CBLK):
                    w = plsc.load_gather(bufs[k], [rvec, colv])
                    accd[k] = accd[k] + w * cn[k][c]
                    accn[k] = accn[k] + w * w
                colv = colv + 1
            acc_d = (accd[0] + accd[1]) + (accd[2] + accd[3])
            acc_n = (accn[0] + accn[1]) + (accn[2] + accn[3])
            inv = _rsqrt16(jnp.maximum(acc_n, jnp.float32(1e-30)))
            sc16 = acc_d * inv
            sc_v[pl.ds(g * L, L)] = sc16
            idx16 = iota + (row0 + g * L)
            better = sc16 > mv
            return (jnp.where(better, sc16, mv),
                    jnp.where(better, idx16, iv))

        mv, iv = plsc.parallel_loop(
            0, GROUPS, 1, unroll=unroll,
            carry=(mvec[...], ivec[...]))(group)
        mvec[...] = mv
        ivec[...] = iv

    def in_copies(row0, bufs, sem):
        return [pltpu.make_async_copy(
            slots_hbm.at[pl.ds(row0, CHUNK), pl.ds(L * k, L)],
            bufs[k].at[:, pl.ds(0, L)], sem) for k in range(CBLK)]

    def in_start(row0, bufs, sem):
        for cp in in_copies(row0, bufs, sem):
            cp.start()

    def in_wait(row0, bufs, sem):
        for cp in in_copies(row0, bufs, sem):
            cp.wait()

    def out_copy(row0, buf, sem):
        return pltpu.make_async_copy(
            buf, scores_hbm.at[pl.ds(row0, CHUNK)], sem)

    base = wid * ROWS_PER_TILE
    in_start(base, in_a, sem_ia)

    def pair(p, carry):
        r0 = base + (2 * p) * CHUNK
        in_wait(r0, in_a, sem_ia)
        in_start(r0 + CHUNK, in_b, sem_ib)

        @pl.when(p > 0)
        def _():
            out_copy(r0 - 2 * CHUNK, sc_a, sem_oa).wait()

        compute_chunk(in_a, sc_a, r0, 5)
        out_copy(r0, sc_a, sem_oa).start()

        in_wait(r0 + CHUNK, in_b, sem_ib)

        @pl.when(p < PAIRS - 1)
        def _():
            in_start(r0 + 2 * CHUNK, in_a, sem_ia)

        @pl.when(p > 0)
        def _():
            out_copy(r0 - CHUNK, sc_b, sem_ob).wait()

        compute_chunk(in_b, sc_b, r0 + CHUNK, 5)
        out_copy(r0 + CHUNK, sc_b, sem_ob).start()
        return carry

    lax.fori_loop(0, PAIRS, pair, 0)
    out_copy(base + (MAIN_PER_TILE - 2) * CHUNK, sc_a, sem_oa).wait()
    out_copy(base + (MAIN_PER_TILE - 1) * CHUNK, sc_b, sem_ob).wait()

    @pl.when(wid < TAIL_CHUNKS)
    def _():
        row0 = MAIN_ROWS + wid * CHUNK
        in_start(row0, in_a, sem_ia)
        in_wait(row0, in_a, sem_ia)
        compute_chunk(in_a, sc_a, row0, 1)
        pltpu.sync_copy(sc_a, scores_hbm.at[pl.ds(row0, CHUNK)])

    cb = tuple(_b16r(x) for x in cn)
    pltpu.async_copy(slots_hbm.at[ivec], rows_v, sem_ia).wait()
    ivv = ivec[...]
    best_s = jnp.float32(-jnp.inf)
    best_i = jnp.int32(0)
    for r in range(L):
        v0 = rows_v[r, pl.ds(0, L)]
        v1 = rows_v[r, pl.ds(L, L)]
        v2 = rows_v[r, pl.ds(2 * L, L)]
        v3 = rows_v[r, pl.ds(3 * L, L)]
        nv = v0 * v0 + v1 * v1 + v2 * v2 + v3 * v3
        n2 = jnp.cumsum(nv)[L - 1]
        invv = _rsqrt16(jnp.full((L,), jnp.maximum(n2, jnp.float32(1e-30)),
                                 jnp.float32))
        u0 = _b16r(v0 * invv)
        u1 = _b16r(v1 * invv)
        u2 = _b16r(v2 * invv)
        u3 = _b16r(v3 * invv)
        sv = u0 * cb[0] + u1 * cb[1] + u2 * cb[2] + u3 * cb[3]
        s_r = jnp.cumsum(sv)[L - 1]
        i_r = ivv[r]
        take = (s_r > best_s) | ((s_r == best_s) & (i_r < best_i))
        best_s = jnp.where(take, s_r, best_s)
        best_i = jnp.where(take, i_r, best_i)
    mvec[...] = jnp.full((L,), best_s, jnp.float32)
    ivec[...] = jnp.full((L,), best_i, jnp.int32)

    pltpu.sync_copy(mvec, pmax_hbm.at[wid])
    pltpu.sync_copy(ivec, pidx_hbm.at[wid])


def _merge_body(pm_ref, pi_ref, o_ref):
    m = pm_ref[...]
    i = pi_ref[...]
    best = jnp.max(m)
    o_ref[0, 0] = jnp.min(jnp.where(m == best, i, jnp.int32(2147483647)))


def _merge(pmax, pidx):
    return pl.pallas_call(
        _merge_body,
        out_shape=jax.ShapeDtypeStruct((1, 1), jnp.int32),
        out_specs=pl.BlockSpec(memory_space=pltpu.SMEM),
    )(pmax, pidx)


@jax.jit
def kernel(candidate, slot_embeds):
    mesh = plsc.VectorSubcoreMesh(core_axis_name="c", subcore_axis_name="s")
    sc_call = pl.kernel(
        _sc_body,
        out_type=[
            jax.ShapeDtypeStruct((N,), jnp.float32),
            jax.ShapeDtypeStruct((NW, L), jnp.float32),
            jax.ShapeDtypeStruct((NW, L), jnp.int32),
        ],
        scratch_types=[
            pltpu.VMEM((D,), jnp.float32),
            pltpu.VMEM((CHUNK, PAD), jnp.float32),
            pltpu.VMEM((CHUNK, PAD), jnp.float32),
            pltpu.VMEM((CHUNK, PAD), jnp.float32),
            pltpu.VMEM((CHUNK, PAD), jnp.float32),
            pltpu.VMEM((CHUNK, PAD), jnp.float32),
            pltpu.VMEM((CHUNK, PAD), jnp.float32),
            pltpu.VMEM((CHUNK, PAD), jnp.float32),
            pltpu.VMEM((CHUNK, PAD), jnp.float32),
            pltpu.VMEM((CHUNK,), jnp.float32),
            pltpu.VMEM((CHUNK,), jnp.float32),
            pltpu.VMEM((L, D), jnp.float32),
            pltpu.VMEM((L,), jnp.float32),
            pltpu.VMEM((L,), jnp.int32),
            pltpu.SemaphoreType.DMA,
            pltpu.SemaphoreType.DMA,
            pltpu.SemaphoreType.DMA,
            pltpu.SemaphoreType.DMA,
        ],
        mesh=mesh,
        compiler_params=pltpu.CompilerParams(needs_layout_passes=False,
                                             use_tc_tiling_on_sc=False),
    )
    scores, pmax, pidx = sc_call(candidate, slot_embeds)
    best = _merge(pmax, pidx)[0, 0]
    return scores, best


# speedup vs baseline: 1.2495x; 1.1313x over previous
import functools

import jax
import jax.numpy as jnp
from jax import lax
from jax.experimental import pallas as pl
from jax.experimental.pallas import tpu as pltpu
from jax.experimental.pallas import tpu_sc as plsc

N = 1_000_000
D = 64
NC = 2
NS = 16
NW = NC * NS
L = 16

CHUNK = 400
PAD = 17
CBLK = 4
MAIN_PER_TILE = 78
PAIRS = MAIN_PER_TILE // 2
ROWS_PER_TILE = CHUNK * MAIN_PER_TILE
MAIN_ROWS = ROWS_PER_TILE * NW
TAIL_CHUNKS = (N - MAIN_ROWS) // CHUNK
GROUPS = CHUNK // L


def _rsqrt16(x):
    xi = plsc.bitcast(x, jnp.int32)
    y = plsc.bitcast(jnp.int32(0x5F3759DF) - (xi >> 1), jnp.float32)
    xh = x * jnp.float32(-0.5)
    for _ in range(3):
        y = y * (jnp.float32(1.5) + xh * y * y)
    return y


def _b16r(x):
    xi = plsc.bitcast(x, jnp.int32)
    return plsc.bitcast((xi + jnp.int32(0x8000)) & jnp.int32(-65536),
                        jnp.float32)


def _sc_body(cand_hbm, slots_hbm, scores_hbm, pmax_hbm, pidx_hbm,
             cand_v, in_a, in_b, sc_a, sc_b, dbuf, nbuf, rows_v, mvec, ivec,
             sem_ia, sem_ib, sem_oa, sem_ob):
    c = lax.axis_index("c")
    s = lax.axis_index("s")
    wid = s * NC + c

    pltpu.sync_copy(cand_hbm, cand_v)
    c0 = cand_v[pl.ds(0, L)]
    c1 = cand_v[pl.ds(L, L)]
    c2 = cand_v[pl.ds(2 * L, L)]
    c3 = cand_v[pl.ds(3 * L, L)]
    cn2 = jnp.sum(c0 * c0 + c1 * c1 + c2 * c2 + c3 * c3)
    inv_c = _rsqrt16(jnp.full((L,), jnp.maximum(cn2, jnp.float32(1e-30)),
                              jnp.float32))
    cn = (c0 * inv_c, c1 * inv_c, c2 * inv_c, c3 * inv_c)

    mvec[...] = jnp.full((L,), -jnp.inf, jnp.float32)
    ivec[...] = jnp.zeros((L,), jnp.int32)
    iota = lax.iota(jnp.int32, L)
    gidx = iota * L + (L - 1)

    def compute_chunk(in_v, sc_v, row0, unroll):
        def group(g, mv_iv):
            mv, iv = mv_iv
            gb = g * (L * L)
            for r in range(L):
                i = g * L + r
                v0 = in_v[i, pl.ds(0, L)]
                v1 = in_v[i, pl.ds(L, L)]
                v2 = in_v[i, pl.ds(2 * L, L)]
                v3 = in_v[i, pl.ds(3 * L, L)]
                sv = (v0 * cn[0] + v1 * cn[1]) + (v2 * cn[2] + v3 * cn[3])
                nv = (v0 * v0 + v1 * v1) + (v2 * v2 + v3 * v3)
                dbuf[pl.ds(gb + r * L, L)] = jnp.cumsum(sv)
                nbuf[pl.ds(gb + r * L, L)] = jnp.cumsum(nv)
            dvec = plsc.load_gather(dbuf, [gidx + gb])
            nvec = plsc.load_gather(nbuf, [gidx + gb])
            inv = _rsqrt16(jnp.maximum(nvec, jnp.float32(1e-30)))
            sc16 = dvec * inv
            sc_v[pl.ds(g * L, L)] = sc16
            idx16 = iota + (row0 + g * L)
            better = sc16 > mv
            return (jnp.where(better, sc16, mv),
                    jnp.where(better, idx16, iv))

        mv, iv = plsc.parallel_loop(
            0, GROUPS, 1, unroll=unroll,
            carry=(mvec[...], ivec[...]))(group)
        mvec[...] = mv
        ivec[...] = iv

    def in_copies(row0, buf, sem):
        return [pltpu.make_async_copy(
            slots_hbm.at[pl.ds(row0, CHUNK), :], buf, sem)]

    def in_start(row0, buf, sem):
        for cp in in_copies(row0, buf, sem):
            cp.start()

    def in_wait(row0, buf, sem):
        for cp in in_copies(row0, buf, sem):
            cp.wait()

    def out_copy(row0, buf, sem):
        return pltpu.make_async_copy(
            buf, scores_hbm.at[pl.ds(row0, CHUNK)], sem)

    base = wid * ROWS_PER_TILE
    in_start(base, in_a, sem_ia)

    def pair(p, carry):
        r0 = base + (2 * p) * CHUNK
        in_wait(r0, in_a, sem_ia)
        in_start(r0 + CHUNK, in_b, sem_ib)

        @pl.when(p > 0)
        def _():
            out_copy(r0 - 2 * CHUNK, sc_a, sem_oa).wait()

        compute_chunk(in_a, sc_a, r0, 5)
        out_copy(r0, sc_a, sem_oa).start()

        in_wait(r0 + CHUNK, in_b, sem_ib)

        @pl.when(p < PAIRS - 1)
        def _():
            in_start(r0 + 2 * CHUNK, in_a, sem_ia)

        @pl.when(p > 0)
        def _():
            out_copy(r0 - CHUNK, sc_b, sem_ob).wait()

        compute_chunk(in_b, sc_b, r0 + CHUNK, 5)
        out_copy(r0 + CHUNK, sc_b, sem_ob).start()
        return carry

    lax.fori_loop(0, PAIRS, pair, 0)
    out_copy(base + (MAIN_PER_TILE - 2) * CHUNK, sc_a, sem_oa).wait()
    out_copy(base + (MAIN_PER_TILE - 1) * CHUNK, sc_b, sem_ob).wait()

    @pl.when(wid < TAIL_CHUNKS)
    def _():
        row0 = MAIN_ROWS + wid * CHUNK
        in_start(row0, in_a, sem_ia)
        in_wait(row0, in_a, sem_ia)
        compute_chunk(in_a, sc_a, row0, 1)
        pltpu.sync_copy(sc_a, scores_hbm.at[pl.ds(row0, CHUNK)])

    cb = tuple(_b16r(x) for x in cn)
    pltpu.async_copy(slots_hbm.at[ivec], rows_v, sem_ia).wait()
    ivv = ivec[...]
    best_s = jnp.float32(-jnp.inf)
    best_i = jnp.int32(0)
    for r in range(L):
        v0 = rows_v[r, pl.ds(0, L)]
        v1 = rows_v[r, pl.ds(L, L)]
        v2 = rows_v[r, pl.ds(2 * L, L)]
        v3 = rows_v[r, pl.ds(3 * L, L)]
        nv = v0 * v0 + v1 * v1 + v2 * v2 + v3 * v3
        n2 = jnp.cumsum(nv)[L - 1]
        invv = _rsqrt16(jnp.full((L,), jnp.maximum(n2, jnp.float32(1e-30)),
                                 jnp.float32))
        u0 = _b16r(v0 * invv)
        u1 = _b16r(v1 * invv)
        u2 = _b16r(v2 * invv)
        u3 = _b16r(v3 * invv)
        sv = u0 * cb[0] + u1 * cb[1] + u2 * cb[2] + u3 * cb[3]
        s_r = jnp.cumsum(sv)[L - 1]
        i_r = ivv[r]
        take = (s_r > best_s) | ((s_r == best_s) & (i_r < best_i))
        best_s = jnp.where(take, s_r, best_s)
        best_i = jnp.where(take, i_r, best_i)
    mvec[...] = jnp.full((L,), best_s, jnp.float32)
    ivec[...] = jnp.full((L,), best_i, jnp.int32)

    pltpu.sync_copy(mvec, pmax_hbm.at[wid])
    pltpu.sync_copy(ivec, pidx_hbm.at[wid])


def _merge_body(pm_ref, pi_ref, o_ref):
    m = pm_ref[...]
    i = pi_ref[...]
    best = jnp.max(m)
    o_ref[0, 0] = jnp.min(jnp.where(m == best, i, jnp.int32(2147483647)))


def _merge(pmax, pidx):
    return pl.pallas_call(
        _merge_body,
        out_shape=jax.ShapeDtypeStruct((1, 1), jnp.int32),
        out_specs=pl.BlockSpec(memory_space=pltpu.SMEM),
    )(pmax, pidx)


@jax.jit
def kernel(candidate, slot_embeds):
    mesh = plsc.VectorSubcoreMesh(core_axis_name="c", subcore_axis_name="s")
    sc_call = pl.kernel(
        _sc_body,
        out_type=[
            jax.ShapeDtypeStruct((N,), jnp.float32),
            jax.ShapeDtypeStruct((NW, L), jnp.float32),
            jax.ShapeDtypeStruct((NW, L), jnp.int32),
        ],
        scratch_types=[
            pltpu.VMEM((D,), jnp.float32),
            pltpu.VMEM((CHUNK, D), jnp.float32),
            pltpu.VMEM((CHUNK, D), jnp.float32),
            pltpu.VMEM((CHUNK,), jnp.float32),
            pltpu.VMEM((CHUNK,), jnp.float32),
            pltpu.VMEM((CHUNK * L,), jnp.float32),
            pltpu.VMEM((CHUNK * L,), jnp.float32),
            pltpu.VMEM((L, D), jnp.float32),
            pltpu.VMEM((L,), jnp.float32),
            pltpu.VMEM((L,), jnp.int32),
            pltpu.SemaphoreType.DMA,
            pltpu.SemaphoreType.DMA,
            pltpu.SemaphoreType.DMA,
            pltpu.SemaphoreType.DMA,
        ],
        mesh=mesh,
        compiler_params=pltpu.CompilerParams(needs_layout_passes=False,
                                             use_tc_tiling_on_sc=False),
    )
    scores, pmax, pidx = sc_call(candidate, slot_embeds)
    best = _merge(pmax, pidx)[0, 0]
    return scores, best


# speedup vs baseline: 1.5371x; 1.2301x over previous
import functools

import jax
import jax.numpy as jnp
from jax import lax
from jax.experimental import pallas as pl
from jax.experimental.pallas import tpu as pltpu
from jax.experimental.pallas import tpu_sc as plsc

N = 1_000_000
D = 64
NC = 2
NS = 16
NW = NC * NS
L = 16

CHUNK = 400
PAD = 17
CBLK = 4
MAIN_PER_TILE = 78
PAIRS = MAIN_PER_TILE // 2
ROWS_PER_TILE = CHUNK * MAIN_PER_TILE
MAIN_ROWS = ROWS_PER_TILE * NW
TAIL_CHUNKS = (N - MAIN_ROWS) // CHUNK
GROUPS = CHUNK // L


def _rsqrt16(x):
    xi = plsc.bitcast(x, jnp.int32)
    y = plsc.bitcast(jnp.int32(0x5F3759DF) - (xi >> 1), jnp.float32)
    xh = x * jnp.float32(-0.5)
    for _ in range(3):
        y = y * (jnp.float32(1.5) + xh * y * y)
    return y


def _b16r(x):
    xi = plsc.bitcast(x, jnp.int32)
    return plsc.bitcast((xi + jnp.int32(0x8000)) & jnp.int32(-65536),
                        jnp.float32)


def _sc_body(cand_hbm, slots_hbm, scores_hbm, pmax_hbm, pidx_hbm,
             cand_v, in_a, in_b, sc_a, sc_b, dbuf, nbuf, rows_v, mvec, ivec,
             sem_ia, sem_ib, sem_oa, sem_ob):
    c = lax.axis_index("c")
    s = lax.axis_index("s")
    wid = s * NC + c

    pltpu.sync_copy(cand_hbm, cand_v)
    c0 = cand_v[pl.ds(0, L)]
    c1 = cand_v[pl.ds(L, L)]
    c2 = cand_v[pl.ds(2 * L, L)]
    c3 = cand_v[pl.ds(3 * L, L)]
    cn2 = jnp.sum(c0 * c0 + c1 * c1 + c2 * c2 + c3 * c3)
    inv_c = _rsqrt16(jnp.full((L,), jnp.maximum(cn2, jnp.float32(1e-30)),
                              jnp.float32))
    cn = (c0 * inv_c, c1 * inv_c, c2 * inv_c, c3 * inv_c)

    mvec[...] = jnp.full((L,), -jnp.inf, jnp.float32)
    ivec[...] = jnp.zeros((L,), jnp.int32)
    iota = lax.iota(jnp.int32, L)
    gidx = iota * L + (L - 1)

    def compute_chunk(in_v, sc_v, row0, unroll):
        def group(g, mv_iv):
            mv, iv = mv_iv
            gb = g * (L * L)
            for r in range(L):
                i = g * L + r
                v0 = in_v[i, pl.ds(0, L)]
                v1 = in_v[i, pl.ds(L, L)]
                v2 = in_v[i, pl.ds(2 * L, L)]
                v3 = in_v[i, pl.ds(3 * L, L)]
                sv = (v0 * cn[0] + v1 * cn[1]) + (v2 * cn[2] + v3 * cn[3])
                nv = (v0 * v0 + v1 * v1) + (v2 * v2 + v3 * v3)
                dbuf[pl.ds(gb + r * L, L)] = jnp.cumsum(sv)
                nbuf[pl.ds(gb + r * L, L)] = jnp.cumsum(nv)
            dvec = plsc.load_gather(dbuf, [gidx + gb])
            nvec = plsc.load_gather(nbuf, [gidx + gb])
            inv = _rsqrt16(jnp.maximum(nvec, jnp.float32(1e-30)))
            sc16 = dvec * inv
            sc_v[pl.ds(g * L, L)] = sc16
            idx16 = iota + (row0 + g * L)
            better = sc16 > mv
            return (jnp.where(better, sc16, mv),
                    jnp.where(better, idx16, iv))

        del unroll
        mv, iv = lax.fori_loop(0, GROUPS, group, (mvec[...], ivec[...]))
        mvec[...] = mv
        ivec[...] = iv

    def in_copies(row0, buf, sem):
        return [pltpu.make_async_copy(
            slots_hbm.at[pl.ds(row0, CHUNK), :], buf, sem)]

    def in_start(row0, buf, sem):
        for cp in in_copies(row0, buf, sem):
            cp.start()

    def in_wait(row0, buf, sem):
        for cp in in_copies(row0, buf, sem):
            cp.wait()

    def out_copy(row0, buf, sem):
        return pltpu.make_async_copy(
            buf, scores_hbm.at[pl.ds(row0, CHUNK)], sem)

    base = wid * ROWS_PER_TILE
    in_start(base, in_a, sem_ia)

    def pair(p, carry):
        r0 = base + (2 * p) * CHUNK
        in_wait(r0, in_a, sem_ia)
        in_start(r0 + CHUNK, in_b, sem_ib)

        @pl.when(p > 0)
        def _():
            out_copy(r0 - 2 * CHUNK, sc_a, sem_oa).wait()

        compute_chunk(in_a, sc_a, r0, 5)
        out_copy(r0, sc_a, sem_oa).start()

        in_wait(r0 + CHUNK, in_b, sem_ib)

        @pl.when(p < PAIRS - 1)
        def _():
            in_start(r0 + 2 * CHUNK, in_a, sem_ia)

        @pl.when(p > 0)
        def _():
            out_copy(r0 - CHUNK, sc_b, sem_ob).wait()

        compute_chunk(in_b, sc_b, r0 + CHUNK, 5)
        out_copy(r0 + CHUNK, sc_b, sem_ob).start()
        return carry

    lax.fori_loop(0, PAIRS, pair, 0)
    out_copy(base + (MAIN_PER_TILE - 2) * CHUNK, sc_a, sem_oa).wait()
    out_copy(base + (MAIN_PER_TILE - 1) * CHUNK, sc_b, sem_ob).wait()

    @pl.when(wid < TAIL_CHUNKS)
    def _():
        row0 = MAIN_ROWS + wid * CHUNK
        in_start(row0, in_a, sem_ia)
        in_wait(row0, in_a, sem_ia)
        compute_chunk(in_a, sc_a, row0, 1)
        pltpu.sync_copy(sc_a, scores_hbm.at[pl.ds(row0, CHUNK)])

    cb = tuple(_b16r(x) for x in cn)
    pltpu.async_copy(slots_hbm.at[ivec], rows_v, sem_ia).wait()
    ivv = ivec[...]
    best_s = jnp.float32(-jnp.inf)
    best_i = jnp.int32(0)
    for r in range(L):
        v0 = rows_v[r, pl.ds(0, L)]
        v1 = rows_v[r, pl.ds(L, L)]
        v2 = rows_v[r, pl.ds(2 * L, L)]
        v3 = rows_v[r, pl.ds(3 * L, L)]
        nv = v0 * v0 + v1 * v1 + v2 * v2 + v3 * v3
        n2 = jnp.cumsum(nv)[L - 1]
        invv = _rsqrt16(jnp.full((L,), jnp.maximum(n2, jnp.float32(1e-30)),
                                 jnp.float32))
        u0 = _b16r(v0 * invv)
        u1 = _b16r(v1 * invv)
        u2 = _b16r(v2 * invv)
        u3 = _b16r(v3 * invv)
        sv = u0 * cb[0] + u1 * cb[1] + u2 * cb[2] + u3 * cb[3]
        s_r = jnp.cumsum(sv)[L - 1]
        i_r = ivv[r]
        take = (s_r > best_s) | ((s_r == best_s) & (i_r < best_i))
        best_s = jnp.where(take, s_r, best_s)
        best_i = jnp.where(take, i_r, best_i)
    mvec[...] = jnp.full((L,), best_s, jnp.float32)
    ivec[...] = jnp.full((L,), best_i, jnp.int32)

    pltpu.sync_copy(mvec, pmax_hbm.at[wid])
    pltpu.sync_copy(ivec, pidx_hbm.at[wid])


def _merge_body(pm_ref, pi_ref, o_ref):
    m = pm_ref[...]
    i = pi_ref[...]
    best = jnp.max(m)
    o_ref[0, 0] = jnp.min(jnp.where(m == best, i, jnp.int32(2147483647)))


def _merge(pmax, pidx):
    return pl.pallas_call(
        _merge_body,
        out_shape=jax.ShapeDtypeStruct((1, 1), jnp.int32),
        out_specs=pl.BlockSpec(memory_space=pltpu.SMEM),
    )(pmax, pidx)


@jax.jit
def kernel(candidate, slot_embeds):
    mesh = plsc.VectorSubcoreMesh(core_axis_name="c", subcore_axis_name="s")
    sc_call = pl.kernel(
        _sc_body,
        out_type=[
            jax.ShapeDtypeStruct((N,), jnp.float32),
            jax.ShapeDtypeStruct((NW, L), jnp.float32),
            jax.ShapeDtypeStruct((NW, L), jnp.int32),
        ],
        scratch_types=[
            pltpu.VMEM((D,), jnp.float32),
            pltpu.VMEM((CHUNK, D), jnp.float32),
            pltpu.VMEM((CHUNK, D), jnp.float32),
            pltpu.VMEM((CHUNK,), jnp.float32),
            pltpu.VMEM((CHUNK,), jnp.float32),
            pltpu.VMEM((CHUNK * L,), jnp.float32),
            pltpu.VMEM((CHUNK * L,), jnp.float32),
            pltpu.VMEM((L, D), jnp.float32),
            pltpu.VMEM((L,), jnp.float32),
            pltpu.VMEM((L,), jnp.int32),
            pltpu.SemaphoreType.DMA,
            pltpu.SemaphoreType.DMA,
            pltpu.SemaphoreType.DMA,
            pltpu.SemaphoreType.DMA,
        ],
        mesh=mesh,
        compiler_params=pltpu.CompilerParams(needs_layout_passes=False,
                                             use_tc_tiling_on_sc=False),
    )
    scores, pmax, pidx = sc_call(candidate, slot_embeds)
    best = _merge(pmax, pidx)[0, 0]
    return scores, best


# speedup vs baseline: 1.8803x; 1.2233x over previous
import functools

import jax
import jax.numpy as jnp
from jax import lax
from jax.experimental import pallas as pl
from jax.experimental.pallas import tpu as pltpu
from jax.experimental.pallas import tpu_sc as plsc

N = 1_000_000
D = 64
NC = 2
NS = 16
NW = NC * NS
L = 16

CHUNK = 400
PAD = 17
CBLK = 4
MAIN_PER_TILE = 78
PAIRS = MAIN_PER_TILE // 2
ROWS_PER_TILE = CHUNK * MAIN_PER_TILE
MAIN_ROWS = ROWS_PER_TILE * NW
TAIL_CHUNKS = (N - MAIN_ROWS) // CHUNK
GROUPS = CHUNK // L


def _rsqrt16(x):
    xi = plsc.bitcast(x, jnp.int32)
    y = plsc.bitcast(jnp.int32(0x5F3759DF) - (xi >> 1), jnp.float32)
    xh = x * jnp.float32(-0.5)
    for _ in range(3):
        y = y * (jnp.float32(1.5) + xh * y * y)
    return y


def _b16r(x):
    xi = plsc.bitcast(x, jnp.int32)
    return plsc.bitcast((xi + jnp.int32(0x8000)) & jnp.int32(-65536),
                        jnp.float32)


def _sc_body(cand_hbm, slots_hbm, scores_hbm, pmax_hbm, pidx_hbm,
             cand_v, in_a, in_b, sc_a, sc_b, dbuf, nbuf, rows_v, mvec, ivec,
             sem_ia, sem_ib, sem_oa, sem_ob):
    c = lax.axis_index("c")
    s = lax.axis_index("s")
    wid = s * NC + c

    pltpu.sync_copy(cand_hbm, cand_v)
    c0 = cand_v[pl.ds(0, L)]
    c1 = cand_v[pl.ds(L, L)]
    c2 = cand_v[pl.ds(2 * L, L)]
    c3 = cand_v[pl.ds(3 * L, L)]
    cn2 = jnp.sum(c0 * c0 + c1 * c1 + c2 * c2 + c3 * c3)
    inv_c = _rsqrt16(jnp.full((L,), jnp.maximum(cn2, jnp.float32(1e-30)),
                              jnp.float32))
    cn = (c0 * inv_c, c1 * inv_c, c2 * inv_c, c3 * inv_c)

    mvec[...] = jnp.full((L,), -jnp.inf, jnp.float32)
    ivec[...] = jnp.zeros((L,), jnp.int32)
    iota = lax.iota(jnp.int32, L)
    gidx = iota * L + (L - 1)

    def compute_chunk(in_v, sc_v, row0, unroll):
        def group(g, mv_iv):
            mv, iv = mv_iv
            gb = g * (L * L)
            for r in range(L):
                i = g * L + r
                v0 = in_v[i, pl.ds(0, L)]
                v1 = in_v[i, pl.ds(L, L)]
                v2 = in_v[i, pl.ds(2 * L, L)]
                v3 = in_v[i, pl.ds(3 * L, L)]
                sv = (v0 * cn[0] + v1 * cn[1]) + (v2 * cn[2] + v3 * cn[3])
                nv = (v0 * v0 + v1 * v1) + (v2 * v2 + v3 * v3)
                dbuf[pl.ds(gb + r * L, L)] = jnp.cumsum(sv)
                nbuf[pl.ds(gb + r * L, L)] = jnp.cumsum(nv)
            dvec = plsc.load_gather(dbuf, [gidx + gb])
            nvec = plsc.load_gather(nbuf, [gidx + gb])
            inv = _rsqrt16(jnp.maximum(nvec, jnp.float32(1e-30)))
            sc16 = dvec * inv
            sc_v[pl.ds(g * L, L)] = sc16
            idx16 = iota + (row0 + g * L)
            better = sc16 > mv
            return (jnp.where(better, sc16, mv),
                    jnp.where(better, idx16, iv))

        del unroll
        mv, iv = lax.fori_loop(0, GROUPS, group, (mvec[...], ivec[...]))
        mvec[...] = mv
        ivec[...] = iv

    def in_copies(row0, buf, sem):
        return [pltpu.make_async_copy(
            slots_hbm.at[pl.ds(row0, CHUNK), :], buf, sem)]

    def in_start(row0, buf, sem):
        for cp in in_copies(row0, buf, sem):
            cp.start()

    def in_wait(row0, buf, sem):
        for cp in in_copies(row0, buf, sem):
            cp.wait()

    def out_copy(row0, buf, sem):
        return pltpu.make_async_copy(
            buf, scores_hbm.at[pl.ds(row0, CHUNK)], sem)

    base = wid * ROWS_PER_TILE
    in_start(base, in_a, sem_ia)

    def pair(p, carry):
        r0 = base + (2 * p) * CHUNK
        in_wait(r0, in_a, sem_ia)
        in_start(r0 + CHUNK, in_b, sem_ib)

        @pl.when(p > 0)
        def _():
            out_copy(r0 - 2 * CHUNK, sc_a, sem_oa).wait()

        compute_chunk(in_a, sc_a, r0, 5)
        out_copy(r0, sc_a, sem_oa).start()

        in_wait(r0 + CHUNK, in_b, sem_ib)

        @pl.when(p < PAIRS - 1)
        def _():
            in_start(r0 + 2 * CHUNK, in_a, sem_ia)

        @pl.when(p > 0)
        def _():
            out_copy(r0 - CHUNK, sc_b, sem_ob).wait()

        compute_chunk(in_b, sc_b, r0 + CHUNK, 5)
        out_copy(r0 + CHUNK, sc_b, sem_ob).start()
        return carry

    lax.fori_loop(0, PAIRS, pair, 0)
    out_copy(base + (MAIN_PER_TILE - 2) * CHUNK, sc_a, sem_oa).wait()
    out_copy(base + (MAIN_PER_TILE - 1) * CHUNK, sc_b, sem_ob).wait()

    @pl.when(wid < TAIL_CHUNKS)
    def _():
        row0 = MAIN_ROWS + wid * CHUNK
        in_start(row0, in_a, sem_ia)
        in_wait(row0, in_a, sem_ia)
        compute_chunk(in_a, sc_a, row0, 1)
        pltpu.sync_copy(sc_a, scores_hbm.at[pl.ds(row0, CHUNK)])

    cb = tuple(_b16r(x) for x in cn)
    ivv = ivec[...]
    best_s = jnp.float32(-jnp.inf)
    best_i = jnp.int32(0)
    for r in range(L):
        i_r0 = ivv[r]
        offa = pl.multiple_of(i_r0 & jnp.int32(-8), 8)
        pltpu.sync_copy(slots_hbm.at[pl.ds(offa, 8), :], rows_v)
        sub = i_r0 & jnp.int32(7)
        v0 = rows_v[sub, pl.ds(0, L)]
        v1 = rows_v[sub, pl.ds(L, L)]
        v2 = rows_v[sub, pl.ds(2 * L, L)]
        v3 = rows_v[sub, pl.ds(3 * L, L)]
        nv = v0 * v0 + v1 * v1 + v2 * v2 + v3 * v3
        n2 = jnp.cumsum(nv)[L - 1]
        invv = _rsqrt16(jnp.full((L,), jnp.maximum(n2, jnp.float32(1e-30)),
                                 jnp.float32))
        u0 = _b16r(v0 * invv)
        u1 = _b16r(v1 * invv)
        u2 = _b16r(v2 * invv)
        u3 = _b16r(v3 * invv)
        sv = u0 * cb[0] + u1 * cb[1] + u2 * cb[2] + u3 * cb[3]
        s_r = jnp.cumsum(sv)[L - 1]
        take = (s_r > best_s) | ((s_r == best_s) & (i_r0 < best_i))
        best_s = jnp.where(take, s_r, best_s)
        best_i = jnp.where(take, i_r0, best_i)
    mvec[...] = jnp.full((L,), best_s, jnp.float32)
    ivec[...] = jnp.full((L,), best_i, jnp.int32)

    pltpu.sync_copy(mvec, pmax_hbm.at[wid])
    pltpu.sync_copy(ivec, pidx_hbm.at[wid])


def _merge_body(pm_ref, pi_ref, o_ref):
    m = pm_ref[...]
    i = pi_ref[...]
    best = jnp.max(m)
    o_ref[0, 0] = jnp.min(jnp.where(m == best, i, jnp.int32(2147483647)))


def _merge(pmax, pidx):
    return pl.pallas_call(
        _merge_body,
        out_shape=jax.ShapeDtypeStruct((1, 1), jnp.int32),
        out_specs=pl.BlockSpec(memory_space=pltpu.SMEM),
    )(pmax, pidx)


@jax.jit
def kernel(candidate, slot_embeds):
    mesh = plsc.VectorSubcoreMesh(core_axis_name="c", subcore_axis_name="s")
    sc_call = pl.kernel(
        _sc_body,
        out_type=[
            jax.ShapeDtypeStruct((N,), jnp.float32),
            jax.ShapeDtypeStruct((NW, L), jnp.float32),
            jax.ShapeDtypeStruct((NW, L), jnp.int32),
        ],
        scratch_types=[
            pltpu.VMEM((D,), jnp.float32),
            pltpu.VMEM((CHUNK, D), jnp.float32),
            pltpu.VMEM((CHUNK, D), jnp.float32),
            pltpu.VMEM((CHUNK,), jnp.float32),
            pltpu.VMEM((CHUNK,), jnp.float32),
            pltpu.VMEM((CHUNK * L,), jnp.float32),
            pltpu.VMEM((CHUNK * L,), jnp.float32),
            pltpu.VMEM((8, D), jnp.float32),
            pltpu.VMEM((L,), jnp.float32),
            pltpu.VMEM((L,), jnp.int32),
            pltpu.SemaphoreType.DMA,
            pltpu.SemaphoreType.DMA,
            pltpu.SemaphoreType.DMA,
            pltpu.SemaphoreType.DMA,
        ],
        mesh=mesh,
        compiler_params=pltpu.CompilerParams(needs_layout_passes=False),
    )
    scores, pmax, pidx = sc_call(candidate, slot_embeds)
    best = _merge(pmax, pidx)[0, 0]
    return scores, best


# speedup vs baseline: 2.9499x; 1.5688x over previous
import functools

import jax
import jax.numpy as jnp
from jax import lax
from jax.experimental import pallas as pl
from jax.experimental.pallas import tpu as pltpu
from jax.experimental.pallas import tpu_sc as plsc

N = 1_000_000
D = 64
NC = 2
NS = 16
NW = NC * NS
L = 16

CHUNK = 400
PAD = 17
CBLK = 4
MAIN_PER_TILE = 78
PAIRS = MAIN_PER_TILE // 2
ROWS_PER_TILE = CHUNK * MAIN_PER_TILE
MAIN_ROWS = ROWS_PER_TILE * NW
TAIL_CHUNKS = (N - MAIN_ROWS) // CHUNK
GROUPS = CHUNK // L


def _rsqrt16(x):
    xi = plsc.bitcast(x, jnp.int32)
    y = plsc.bitcast(jnp.int32(0x5F3759DF) - (xi >> 1), jnp.float32)
    xh = x * jnp.float32(-0.5)
    for _ in range(3):
        y = y * (jnp.float32(1.5) + xh * y * y)
    return y


def _b16r(x):
    xi = plsc.bitcast(x, jnp.int32)
    return plsc.bitcast((xi + jnp.int32(0x8000)) & jnp.int32(-65536),
                        jnp.float32)


def _sc_body(cand_hbm, slots_hbm, scores_hbm, pmax_hbm, pidx_hbm,
             cand_v, in_a, in_b, sc_a, sc_b, dbuf, nbuf, rows_v, mvec, ivec,
             sem_ia, sem_ib, sem_oa, sem_ob):
    c = lax.axis_index("c")
    s = lax.axis_index("s")
    wid = s * NC + c

    pltpu.sync_copy(cand_hbm, cand_v)
    c0 = cand_v[pl.ds(0, L)]
    c1 = cand_v[pl.ds(L, L)]
    c2 = cand_v[pl.ds(2 * L, L)]
    c3 = cand_v[pl.ds(3 * L, L)]
    cn2 = jnp.sum(c0 * c0 + c1 * c1 + c2 * c2 + c3 * c3)
    inv_c = _rsqrt16(jnp.full((L,), jnp.maximum(cn2, jnp.float32(1e-30)),
                              jnp.float32))
    cn = (c0 * inv_c, c1 * inv_c, c2 * inv_c, c3 * inv_c)

    mvec[...] = jnp.full((L,), -jnp.inf, jnp.float32)
    ivec[...] = jnp.zeros((L,), jnp.int32)
    iota = lax.iota(jnp.int32, L)
    rots = [(iota + jnp.int32(k)) & jnp.int32(L - 1) for k in (8, 4, 2, 1)]

    def hsum(x):
        for rv in rots:
            x = x + jnp.take_along_axis(x, rv, axis=0)
        return x

    def compute_chunk(in_v, sc_v, row0, unroll):
        def group(g, mv_iv):
            mv, iv = mv_iv
            dvec = jnp.zeros((L,), jnp.float32)
            nvec = jnp.zeros((L,), jnp.float32)
            for r in range(L):
                i = g * L + r
                v0 = in_v[i, pl.ds(0, L)]
                v1 = in_v[i, pl.ds(L, L)]
                v2 = in_v[i, pl.ds(2 * L, L)]
                v3 = in_v[i, pl.ds(3 * L, L)]
                sv = (v0 * cn[0] + v1 * cn[1]) + (v2 * cn[2] + v3 * cn[3])
                nv = (v0 * v0 + v1 * v1) + (v2 * v2 + v3 * v3)
                lane_r = iota == jnp.int32(r)
                dvec = jnp.where(lane_r, hsum(sv), dvec)
                nvec = jnp.where(lane_r, hsum(nv), nvec)
            inv = _rsqrt16(jnp.maximum(nvec, jnp.float32(1e-30)))
            sc16 = dvec * inv
            sc_v[pl.ds(g * L, L)] = sc16
            idx16 = iota + (row0 + g * L)
            better = sc16 > mv
            return (jnp.where(better, sc16, mv),
                    jnp.where(better, idx16, iv))

        del unroll
        mv, iv = lax.fori_loop(0, GROUPS, group, (mvec[...], ivec[...]))
        mvec[...] = mv
        ivec[...] = iv

    def in_copies(row0, buf, sem):
        return [pltpu.make_async_copy(
            slots_hbm.at[pl.ds(row0, CHUNK), :], buf, sem)]

    def in_start(row0, buf, sem):
        for cp in in_copies(row0, buf, sem):
            cp.start()

    def in_wait(row0, buf, sem):
        for cp in in_copies(row0, buf, sem):
            cp.wait()

    def out_copy(row0, buf, sem):
        return pltpu.make_async_copy(
            buf, scores_hbm.at[pl.ds(row0, CHUNK)], sem)

    base = wid * ROWS_PER_TILE
    in_start(base, in_a, sem_ia)

    def pair(p, carry):
        r0 = base + (2 * p) * CHUNK
        in_wait(r0, in_a, sem_ia)
        in_start(r0 + CHUNK, in_b, sem_ib)

        @pl.when(p > 0)
        def _():
            out_copy(r0 - 2 * CHUNK, sc_a, sem_oa).wait()

        compute_chunk(in_a, sc_a, r0, 5)
        out_copy(r0, sc_a, sem_oa).start()

        in_wait(r0 + CHUNK, in_b, sem_ib)

        @pl.when(p < PAIRS - 1)
        def _():
            in_start(r0 + 2 * CHUNK, in_a, sem_ia)

        @pl.when(p > 0)
        def _():
            out_copy(r0 - CHUNK, sc_b, sem_ob).wait()

        compute_chunk(in_b, sc_b, r0 + CHUNK, 5)
        out_copy(r0 + CHUNK, sc_b, sem_ob).start()
        return carry

    lax.fori_loop(0, PAIRS, pair, 0)
    out_copy(base + (MAIN_PER_TILE - 2) * CHUNK, sc_a, sem_oa).wait()
    out_copy(base + (MAIN_PER_TILE - 1) * CHUNK, sc_b, sem_ob).wait()

    @pl.when(wid < TAIL_CHUNKS)
    def _():
        row0 = MAIN_ROWS + wid * CHUNK
        in_start(row0, in_a, sem_ia)
        in_wait(row0, in_a, sem_ia)
        compute_chunk(in_a, sc_a, row0, 1)
        pltpu.sync_copy(sc_a, scores_hbm.at[pl.ds(row0, CHUNK)])

    cb = tuple(_b16r(x) for x in cn)
    ivv = ivec[...]
    best_s = jnp.float32(-jnp.inf)
    best_i = jnp.int32(0)
    for r in range(L):
        i_r0 = ivv[r]
        offa = pl.multiple_of(i_r0 & jnp.int32(-8), 8)
        pltpu.sync_copy(slots_hbm.at[pl.ds(offa, 8), :], rows_v)
        sub = i_r0 & jnp.int32(7)
        v0 = rows_v[sub, pl.ds(0, L)]
        v1 = rows_v[sub, pl.ds(L, L)]
        v2 = rows_v[sub, pl.ds(2 * L, L)]
        v3 = rows_v[sub, pl.ds(3 * L, L)]
        nv = v0 * v0 + v1 * v1 + v2 * v2 + v3 * v3
        n2 = jnp.cumsum(nv)[L - 1]
        invv = _rsqrt16(jnp.full((L,), jnp.maximum(n2, jnp.float32(1e-30)),
                                 jnp.float32))
        u0 = _b16r(v0 * invv)
        u1 = _b16r(v1 * invv)
        u2 = _b16r(v2 * invv)
        u3 = _b16r(v3 * invv)
        sv = u0 * cb[0] + u1 * cb[1] + u2 * cb[2] + u3 * cb[3]
        s_r = jnp.cumsum(sv)[L - 1]
        take = (s_r > best_s) | ((s_r == best_s) & (i_r0 < best_i))
        best_s = jnp.where(take, s_r, best_s)
        best_i = jnp.where(take, i_r0, best_i)
    mvec[...] = jnp.full((L,), best_s, jnp.float32)
    ivec[...] = jnp.full((L,), best_i, jnp.int32)

    pltpu.sync_copy(mvec, pmax_hbm.at[wid])
    pltpu.sync_copy(ivec, pidx_hbm.at[wid])


def _merge_body(pm_ref, pi_ref, o_ref):
    m = pm_ref[...]
    i = pi_ref[...]
    best = jnp.max(m)
    o_ref[0, 0] = jnp.min(jnp.where(m == best, i, jnp.int32(2147483647)))


def _merge(pmax, pidx):
    return pl.pallas_call(
        _merge_body,
        out_shape=jax.ShapeDtypeStruct((1, 1), jnp.int32),
        out_specs=pl.BlockSpec(memory_space=pltpu.SMEM),
    )(pmax, pidx)


@jax.jit
def kernel(candidate, slot_embeds):
    mesh = plsc.VectorSubcoreMesh(core_axis_name="c", subcore_axis_name="s")
    sc_call = pl.kernel(
        _sc_body,
        out_type=[
            jax.ShapeDtypeStruct((N,), jnp.float32),
            jax.ShapeDtypeStruct((NW, L), jnp.float32),
            jax.ShapeDtypeStruct((NW, L), jnp.int32),
        ],
        scratch_types=[
            pltpu.VMEM((D,), jnp.float32),
            pltpu.VMEM((CHUNK, D), jnp.float32),
            pltpu.VMEM((CHUNK, D), jnp.float32),
            pltpu.VMEM((CHUNK,), jnp.float32),
            pltpu.VMEM((CHUNK,), jnp.float32),
            pltpu.VMEM((CHUNK * L,), jnp.float32),
            pltpu.VMEM((CHUNK * L,), jnp.float32),
            pltpu.VMEM((8, D), jnp.float32),
            pltpu.VMEM((L,), jnp.float32),
            pltpu.VMEM((L,), jnp.int32),
            pltpu.SemaphoreType.DMA,
            pltpu.SemaphoreType.DMA,
            pltpu.SemaphoreType.DMA,
            pltpu.SemaphoreType.DMA,
        ],
        mesh=mesh,
        compiler_params=pltpu.CompilerParams(needs_layout_passes=False),
    )
    scores, pmax, pidx = sc_call(candidate, slot_embeds)
    best = _merge(pmax, pidx)[0, 0]
    return scores, best


# speedup vs baseline: 2.9605x; 1.0036x over previous
import functools

import jax
import jax.numpy as jnp
from jax import lax
from jax.experimental import pallas as pl
from jax.experimental.pallas import tpu as pltpu
from jax.experimental.pallas import tpu_sc as plsc

N = 1_000_000
D = 64
NC = 2
NS = 16
NW = NC * NS
L = 16

CHUNK = 400
PAD = 17
CBLK = 4
MAIN_PER_TILE = 78
PAIRS = MAIN_PER_TILE // 2
ROWS_PER_TILE = CHUNK * MAIN_PER_TILE
MAIN_ROWS = ROWS_PER_TILE * NW
TAIL_CHUNKS = (N - MAIN_ROWS) // CHUNK
GROUPS = CHUNK // L


def _rsqrt16(x):
    xi = plsc.bitcast(x, jnp.int32)
    y = plsc.bitcast(jnp.int32(0x5F3759DF) - (xi >> 1), jnp.float32)
    xh = x * jnp.float32(-0.5)
    for _ in range(3):
        y = y * (jnp.float32(1.5) + xh * y * y)
    return y


def _b16r(x):
    xi = plsc.bitcast(x, jnp.int32)
    return plsc.bitcast((xi + jnp.int32(0x8000)) & jnp.int32(-65536),
                        jnp.float32)


def _sc_body(cand_hbm, slots_hbm, scores_hbm, pmax_hbm, pidx_hbm,
             cand_v, in_a, in_b, sc_a, sc_b, dbuf, nbuf, rows_v, mvec, ivec,
             sem_ia, sem_ib, sem_oa, sem_ob):
    c = lax.axis_index("c")
    s = lax.axis_index("s")
    wid = s * NC + c

    pltpu.sync_copy(cand_hbm, cand_v)
    c0 = cand_v[pl.ds(0, L)]
    c1 = cand_v[pl.ds(L, L)]
    c2 = cand_v[pl.ds(2 * L, L)]
    c3 = cand_v[pl.ds(3 * L, L)]
    cn2 = jnp.sum(c0 * c0 + c1 * c1 + c2 * c2 + c3 * c3)
    inv_c = _rsqrt16(jnp.full((L,), jnp.maximum(cn2, jnp.float32(1e-30)),
                              jnp.float32))
    cn = (c0 * inv_c, c1 * inv_c, c2 * inv_c, c3 * inv_c)

    mvec[...] = jnp.full((L,), -jnp.inf, jnp.float32)
    ivec[...] = jnp.zeros((L,), jnp.int32)
    iota = lax.iota(jnp.int32, L)
    rots = [(iota + jnp.int32(k)) & jnp.int32(L - 1) for k in (8, 4, 2, 1)]

    def hsum(x):
        for rv in rots:
            x = x + jnp.take_along_axis(x, rv, axis=0)
        return x

    def compute_chunk(in_v, sc_v, row0, unroll):
        def group(g, mv_iv):
            mv, iv = mv_iv
            svs = []
            nvs = []
            for r in range(L):
                i = g * L + r
                v0 = in_v[i, pl.ds(0, L)]
                v1 = in_v[i, pl.ds(L, L)]
                v2 = in_v[i, pl.ds(2 * L, L)]
                v3 = in_v[i, pl.ds(3 * L, L)]
                svs.append((v0 * cn[0] + v1 * cn[1])
                           + (v2 * cn[2] + v3 * cn[3]))
                nvs.append((v0 * v0 + v1 * v1) + (v2 * v2 + v3 * v3))
            chains = svs + nvs
            for rv in rots:
                chains = [x + jnp.take_along_axis(x, rv, axis=0)
                          for x in chains]
            dvec = jnp.zeros((L,), jnp.float32)
            nvec = jnp.zeros((L,), jnp.float32)
            for r in range(L):
                lane_r = iota == jnp.int32(r)
                dvec = jnp.where(lane_r, chains[r], dvec)
                nvec = jnp.where(lane_r, chains[L + r], nvec)
            inv = _rsqrt16(jnp.maximum(nvec, jnp.float32(1e-30)))
            sc16 = dvec * inv
            sc_v[pl.ds(g * L, L)] = sc16
            idx16 = iota + (row0 + g * L)
            better = sc16 > mv
            return (jnp.where(better, sc16, mv),
                    jnp.where(better, idx16, iv))

        del unroll
        mv, iv = lax.fori_loop(0, GROUPS, group, (mvec[...], ivec[...]))
        mvec[...] = mv
        ivec[...] = iv

    def in_copies(row0, buf, sem):
        return [pltpu.make_async_copy(
            slots_hbm.at[pl.ds(row0, CHUNK), :], buf, sem)]

    def in_start(row0, buf, sem):
        for cp in in_copies(row0, buf, sem):
            cp.start()

    def in_wait(row0, buf, sem):
        for cp in in_copies(row0, buf, sem):
            cp.wait()

    def out_copy(row0, buf, sem):
        return pltpu.make_async_copy(
            buf, scores_hbm.at[pl.ds(row0, CHUNK)], sem)

    base = wid * ROWS_PER_TILE
    in_start(base, in_a, sem_ia)

    def pair(p, carry):
        r0 = base + (2 * p) * CHUNK
        in_wait(r0, in_a, sem_ia)
        in_start(r0 + CHUNK, in_b, sem_ib)

        @pl.when(p > 0)
        def _():
            out_copy(r0 - 2 * CHUNK, sc_a, sem_oa).wait()

        compute_chunk(in_a, sc_a, r0, 5)
        out_copy(r0, sc_a, sem_oa).start()

        in_wait(r0 + CHUNK, in_b, sem_ib)

        @pl.when(p < PAIRS - 1)
        def _():
            in_start(r0 + 2 * CHUNK, in_a, sem_ia)

        @pl.when(p > 0)
        def _():
            out_copy(r0 - CHUNK, sc_b, sem_ob).wait()

        compute_chunk(in_b, sc_b, r0 + CHUNK, 5)
        out_copy(r0 + CHUNK, sc_b, sem_ob).start()
        return carry

    lax.fori_loop(0, PAIRS, pair, 0)
    out_copy(base + (MAIN_PER_TILE - 2) * CHUNK, sc_a, sem_oa).wait()
    out_copy(base + (MAIN_PER_TILE - 1) * CHUNK, sc_b, sem_ob).wait()

    @pl.when(wid < TAIL_CHUNKS)
    def _():
        row0 = MAIN_ROWS + wid * CHUNK
        in_start(row0, in_a, sem_ia)
        in_wait(row0, in_a, sem_ia)
        compute_chunk(in_a, sc_a, row0, 1)
        pltpu.sync_copy(sc_a, scores_hbm.at[pl.ds(row0, CHUNK)])

    cb = tuple(_b16r(x) for x in cn)
    ivv = ivec[...]
    best_s = jnp.float32(-jnp.inf)
    best_i = jnp.int32(0)
    for r in range(L):
        i_r0 = ivv[r]
        offa = pl.multiple_of(i_r0 & jnp.int32(-8), 8)
        pltpu.sync_copy(slots_hbm.at[pl.ds(offa, 8), :], rows_v)
        sub = i_r0 & jnp.int32(7)
        v0 = rows_v[sub, pl.ds(0, L)]
        v1 = rows_v[sub, pl.ds(L, L)]
        v2 = rows_v[sub, pl.ds(2 * L, L)]
        v3 = rows_v[sub, pl.ds(3 * L, L)]
        nv = v0 * v0 + v1 * v1 + v2 * v2 + v3 * v3
        n2 = jnp.cumsum(nv)[L - 1]
        invv = _rsqrt16(jnp.full((L,), jnp.maximum(n2, jnp.float32(1e-30)),
                                 jnp.float32))
        u0 = _b16r(v0 * invv)
        u1 = _b16r(v1 * invv)
        u2 = _b16r(v2 * invv)
        u3 = _b16r(v3 * invv)
        sv = u0 * cb[0] + u1 * cb[1] + u2 * cb[2] + u3 * cb[3]
        s_r = jnp.cumsum(sv)[L - 1]
        take = (s_r > best_s) | ((s_r == best_s) & (i_r0 < best_i))
        best_s = jnp.where(take, s_r, best_s)
        best_i = jnp.where(take, i_r0, best_i)
    mvec[...] = jnp.full((L,), best_s, jnp.float32)
    ivec[...] = jnp.full((L,), best_i, jnp.int32)

    pltpu.sync_copy(mvec, pmax_hbm.at[wid])
    pltpu.sync_copy(ivec, pidx_hbm.at[wid])


def _merge_body(pm_ref, pi_ref, o_ref):
    m = pm_ref[...]
    i = pi_ref[...]
    best = jnp.max(m)
    o_ref[0, 0] = jnp.min(jnp.where(m == best, i, jnp.int32(2147483647)))


def _merge(pmax, pidx):
    return pl.pallas_call(
        _merge_body,
        out_shape=jax.ShapeDtypeStruct((1, 1), jnp.int32),
        out_specs=pl.BlockSpec(memory_space=pltpu.SMEM),
    )(pmax, pidx)


@jax.jit
def kernel(candidate, slot_embeds):
    mesh = plsc.VectorSubcoreMesh(core_axis_name="c", subcore_axis_name="s")
    sc_call = pl.kernel(
        _sc_body,
        out_type=[
            jax.ShapeDtypeStruct((N,), jnp.float32),
            jax.ShapeDtypeStruct((NW, L), jnp.float32),
            jax.ShapeDtypeStruct((NW, L), jnp.int32),
        ],
        scratch_types=[
            pltpu.VMEM((D,), jnp.float32),
            pltpu.VMEM((CHUNK, D), jnp.float32),
            pltpu.VMEM((CHUNK, D), jnp.float32),
            pltpu.VMEM((CHUNK,), jnp.float32),
            pltpu.VMEM((CHUNK,), jnp.float32),
            pltpu.VMEM((CHUNK * L,), jnp.float32),
            pltpu.VMEM((CHUNK * L,), jnp.float32),
            pltpu.VMEM((8, D), jnp.float32),
            pltpu.VMEM((L,), jnp.float32),
            pltpu.VMEM((L,), jnp.int32),
            pltpu.SemaphoreType.DMA,
            pltpu.SemaphoreType.DMA,
            pltpu.SemaphoreType.DMA,
            pltpu.SemaphoreType.DMA,
        ],
        mesh=mesh,
        compiler_params=pltpu.CompilerParams(needs_layout_passes=False),
    )
    scores, pmax, pidx = sc_call(candidate, slot_embeds)
    best = _merge(pmax, pidx)[0, 0]
    return scores, best
